# Initial kernel scaffold; baseline (speedup 1.0000x reference)
import jax
import jax.numpy as jnp
from jax.experimental import pallas as pl


def kernel(emb_mi, emb_di, meta_mdm, meta_mdmdm, meta_dmd, meta_dmdmd, W_mdm, bfc_mdm, bias_mdm, p_mdm, W_mdmdm, bfc_mdmdm, bias_mdmdm, p_mdmdm, W_dmd, bfc_dmd, bias_dmd, p_dmd, W_dmdmd, bfc_dmdmd, bias_dmdmd, p_dmdmd, W_sla, b_sla, a_sla):
    raise NotImplementedError("write your pallas kernel here")


# speedup vs baseline: 1.0808x; 1.0808x over previous
import functools

import jax
import jax.numpy as jnp
from jax.experimental import pallas as pl
from jax.experimental.pallas import tpu as pltpu

N = 4096
D = 512
BM = 256


def _view_body(emb_ref, wt_ref, bfc_ref, bias_ref, p_ref, wslat_ref, bsla_ref,
               asla_ref, meta_ref, out_ref, logit_ref, h_scr, acc_scr):
    i = pl.program_id(0)
    nsteps = pl.num_programs(0)

    @pl.when(i == 0)
    def _init():
        h = jnp.dot(emb_ref[...], wt_ref[...],
                    preferred_element_type=jnp.float32) + bfc_ref[...]
        h_scr[...] = h.astype(jnp.bfloat16)
        acc_scr[...] = jnp.zeros_like(acc_scr)

    out = jnp.dot(meta_ref[...].astype(jnp.bfloat16), h_scr[...],
                  preferred_element_type=jnp.float32) + bias_ref[...]
    out = jnp.where(out >= 0, out, p_ref[0, 0] * out)
    out_ref[...] = out

    s = jnp.tanh(jnp.dot(out.astype(jnp.bfloat16), wslat_ref[...],
                         preferred_element_type=jnp.float32) + bsla_ref[...])
    acc_scr[...] += jnp.sum(s, axis=0, keepdims=True)

    @pl.when(i == nsteps - 1)
    def _fin():
        feat = acc_scr[...] * (1.0 / N)
        logit_ref[...] = jnp.sum(asla_ref[...] * feat, axis=1, keepdims=True)


def _view_call(emb_bf16, wt_bf16, bfc, bias, p, wslat_bf16, bsla, asla, meta):
    grid = (N // BM,)
    out, logit = pl.pallas_call(
        _view_body,
        grid=grid,
        in_specs=[
            pl.BlockSpec((N, D), lambda i: (0, 0)),
            pl.BlockSpec((D, D), lambda i: (0, 0)),
            pl.BlockSpec((1, D), lambda i: (0, 0)),
            pl.BlockSpec((1, D), lambda i: (0, 0)),
            pl.BlockSpec((1, 1), lambda i: (0, 0)),
            pl.BlockSpec((D, D), lambda i: (0, 0)),
            pl.BlockSpec((1, D), lambda i: (0, 0)),
            pl.BlockSpec((1, D), lambda i: (0, 0)),
            pl.BlockSpec((BM, N), lambda i: (i, 0)),
        ],
        out_specs=[
            pl.BlockSpec((BM, D), lambda i: (i, 0)),
            pl.BlockSpec((1, 1), lambda i: (0, 0)),
        ],
        out_shape=[
            jax.ShapeDtypeStruct((N, D), jnp.float32),
            jax.ShapeDtypeStruct((1, 1), jnp.float32),
        ],
        scratch_shapes=[
            pltpu.VMEM((N, D), jnp.bfloat16),
            pltpu.VMEM((1, D), jnp.float32),
        ],
    )(emb_bf16, wt_bf16, bfc, bias, p, wslat_bf16, bsla, asla, meta)
    return out, logit


def _combine_body(logits_ref, v1_ref, v2_ref, v3_ref, v4_ref, o1_ref, o2_ref):
    l = logits_ref[...]
    la = l[:, 0:2]
    lb = l[:, 2:4]
    ea = jnp.exp(la - jnp.max(la))
    eb = jnp.exp(lb - jnp.max(lb))
    ba = ea / jnp.sum(ea)
    bb = eb / jnp.sum(eb)
    o1_ref[...] = v1_ref[...] * ba[0:1, 0:1] + v2_ref[...] * ba[0:1, 1:2]
    o2_ref[...] = v3_ref[...] * bb[0:1, 0:1] + v4_ref[...] * bb[0:1, 1:2]


def _combine_call(logits, v1, v2, v3, v4):
    grid = (N // BM,)
    blk = pl.BlockSpec((BM, D), lambda i: (i, 0))
    return pl.pallas_call(
        _combine_body,
        grid=grid,
        in_specs=[pl.BlockSpec((1, 4), lambda i: (0, 0)), blk, blk, blk, blk],
        out_specs=[blk, blk],
        out_shape=[
            jax.ShapeDtypeStruct((N, D), jnp.float32),
            jax.ShapeDtypeStruct((N, D), jnp.float32),
        ],
    )(logits, v1, v2, v3, v4)


@jax.jit
def kernel(emb_mi, emb_di, meta_mdm, meta_mdmdm, meta_dmd, meta_dmdmd,
           W_mdm, bfc_mdm, bias_mdm, p_mdm,
           W_mdmdm, bfc_mdmdm, bias_mdmdm, p_mdmdm,
           W_dmd, bfc_dmd, bias_dmd, p_dmd,
           W_dmdmd, bfc_dmdmd, bias_dmdmd, p_dmdmd,
           W_sla, b_sla, a_sla):
    emb_mi_bf = emb_mi.astype(jnp.bfloat16)
    emb_di_bf = emb_di.astype(jnp.bfloat16)
    wslat = W_sla.T.astype(jnp.bfloat16)
    bsla = b_sla.reshape(1, D)
    asla = a_sla.reshape(1, D)

    views = []
    logits = []
    for emb_bf, meta, W, bfc, bias, p in (
            (emb_mi_bf, meta_mdm, W_mdm, bfc_mdm, bias_mdm, p_mdm),
            (emb_mi_bf, meta_mdmdm, W_mdmdm, bfc_mdmdm, bias_mdmdm, p_mdmdm),
            (emb_di_bf, meta_dmd, W_dmd, bfc_dmd, bias_dmd, p_dmd),
            (emb_di_bf, meta_dmdmd, W_dmdmd, bfc_dmdmd, bias_dmdmd, p_dmdmd)):
        v, lg = _view_call(emb_bf, W.T.astype(jnp.bfloat16),
                           bfc.reshape(1, D), bias.reshape(1, D),
                           p.reshape(1, 1), wslat, bsla, asla, meta)
        views.append(v)
        logits.append(lg)

    logits4 = jnp.concatenate(logits, axis=1)
    out_mi, out_di = _combine_call(logits4, *views)
    return out_mi, out_di


# speedup vs baseline: 1.1967x; 1.1073x over previous
import jax
import jax.numpy as jnp
from jax.experimental import pallas as pl
from jax.experimental.pallas import tpu as pltpu

N = 4096
D = 512
BM = 256
NB = N // BM


def _pair_body(emb_ref, w1t_ref, w2t_ref, bfc1_ref, bfc2_ref, bias1_ref,
               bias2_ref, p1_ref, p2_ref, wslat_ref, bsla_ref, asla_ref,
               meta1_ref, meta2_ref, out_ref,
               h_scr, v1_scr, v2_scr, acc1_scr, acc2_scr):
    i = pl.program_id(0)

    @pl.when(i == 0)
    def _h1():
        h = jnp.dot(emb_ref[...], w1t_ref[...],
                    preferred_element_type=jnp.float32) + bfc1_ref[...]
        h_scr[...] = h.astype(jnp.bfloat16)
        acc1_scr[...] = jnp.zeros_like(acc1_scr)

    @pl.when(i == NB)
    def _h2():
        h = jnp.dot(emb_ref[...], w2t_ref[...],
                    preferred_element_type=jnp.float32) + bfc2_ref[...]
        h_scr[...] = h.astype(jnp.bfloat16)
        acc2_scr[...] = jnp.zeros_like(acc2_scr)

    @pl.when(i < NB)
    def _view1():
        out = jnp.dot(meta1_ref[...].astype(jnp.bfloat16), h_scr[...],
                      preferred_element_type=jnp.float32) + bias1_ref[...]
        vb = jnp.where(out >= 0, out, p1_ref[0, 0] * out).astype(jnp.bfloat16)
        v1_scr[pl.ds(i * BM, BM), :] = vb
        s = jnp.tanh(jnp.dot(vb, wslat_ref[...],
                             preferred_element_type=jnp.float32) + bsla_ref[...])
        acc1_scr[...] += jnp.sum(s, axis=0, keepdims=True)

    @pl.when(jnp.logical_and(i >= NB, i < 2 * NB))
    def _view2():
        j = i - NB
        out = jnp.dot(meta2_ref[...].astype(jnp.bfloat16), h_scr[...],
                      preferred_element_type=jnp.float32) + bias2_ref[...]
        vb = jnp.where(out >= 0, out, p2_ref[0, 0] * out).astype(jnp.bfloat16)
        v2_scr[pl.ds(j * BM, BM), :] = vb
        s = jnp.tanh(jnp.dot(vb, wslat_ref[...],
                             preferred_element_type=jnp.float32) + bsla_ref[...])
        acc2_scr[...] += jnp.sum(s, axis=0, keepdims=True)

    @pl.when(i >= 2 * NB)
    def _combine():
        j = i - 2 * NB
        la = jnp.sum(asla_ref[...] * acc1_scr[...] * (1.0 / N),
                     axis=1, keepdims=True)
        lb = jnp.sum(asla_ref[...] * acc2_scr[...] * (1.0 / N),
                     axis=1, keepdims=True)
        m = jnp.maximum(la, lb)
        ea = jnp.exp(la - m)
        eb = jnp.exp(lb - m)
        inv = 1.0 / (ea + eb)
        b1 = ea * inv
        b2 = eb * inv
        v1 = v1_scr[pl.ds(j * BM, BM), :].astype(jnp.float32)
        v2 = v2_scr[pl.ds(j * BM, BM), :].astype(jnp.float32)
        out_ref[...] = v1 * b1 + v2 * b2


def _pair_call(emb_bf, w1t, w2t, bfc1, bfc2, bias1, bias2, p1, p2,
               wslat, bsla, asla, meta1, meta2):
    const = lambda i: (0, 0)
    return pl.pallas_call(
        _pair_body,
        grid=(3 * NB,),
        in_specs=[
            pl.BlockSpec((N, D), const),
            pl.BlockSpec((D, D), const),
            pl.BlockSpec((D, D), const),
            pl.BlockSpec((1, D), const),
            pl.BlockSpec((1, D), const),
            pl.BlockSpec((1, D), const),
            pl.BlockSpec((1, D), const),
            pl.BlockSpec((1, 1), const),
            pl.BlockSpec((1, 1), const),
            pl.BlockSpec((D, D), const),
            pl.BlockSpec((1, D), const),
            pl.BlockSpec((1, D), const),
            pl.BlockSpec((BM, N), lambda i: (jnp.minimum(i, NB - 1), 0)),
            pl.BlockSpec((BM, N),
                         lambda i: (jnp.clip(i - NB, 0, NB - 1), 0)),
        ],
        out_specs=pl.BlockSpec((BM, D),
                               lambda i: (jnp.clip(i - 2 * NB, 0, NB - 1), 0)),
        out_shape=jax.ShapeDtypeStruct((N, D), jnp.float32),
        scratch_shapes=[
            pltpu.VMEM((N, D), jnp.bfloat16),
            pltpu.VMEM((N, D), jnp.bfloat16),
            pltpu.VMEM((N, D), jnp.bfloat16),
            pltpu.VMEM((1, D), jnp.float32),
            pltpu.VMEM((1, D), jnp.float32),
        ],
    )(emb_bf, w1t, w2t, bfc1, bfc2, bias1, bias2, p1, p2, wslat, bsla, asla,
      meta1, meta2)


@jax.jit
def kernel(emb_mi, emb_di, meta_mdm, meta_mdmdm, meta_dmd, meta_dmdmd,
           W_mdm, bfc_mdm, bias_mdm, p_mdm,
           W_mdmdm, bfc_mdmdm, bias_mdmdm, p_mdmdm,
           W_dmd, bfc_dmd, bias_dmd, p_dmd,
           W_dmdmd, bfc_dmdmd, bias_dmdmd, p_dmdmd,
           W_sla, b_sla, a_sla):
    wslat = W_sla.T.astype(jnp.bfloat16)
    bsla = b_sla.reshape(1, D)
    asla = a_sla.reshape(1, D)

    out_mi = _pair_call(
        emb_mi.astype(jnp.bfloat16),
        W_mdm.T.astype(jnp.bfloat16), W_mdmdm.T.astype(jnp.bfloat16),
        bfc_mdm.reshape(1, D), bfc_mdmdm.reshape(1, D),
        bias_mdm.reshape(1, D), bias_mdmdm.reshape(1, D),
        p_mdm.reshape(1, 1), p_mdmdm.reshape(1, 1),
        wslat, bsla, asla, meta_mdm, meta_mdmdm)
    out_di = _pair_call(
        emb_di.astype(jnp.bfloat16),
        W_dmd.T.astype(jnp.bfloat16), W_dmdmd.T.astype(jnp.bfloat16),
        bfc_dmd.reshape(1, D), bfc_dmdmd.reshape(1, D),
        bias_dmd.reshape(1, D), bias_dmdmd.reshape(1, D),
        p_dmd.reshape(1, 1), p_dmdmd.reshape(1, 1),
        wslat, bsla, asla, meta_dmd, meta_dmdmd)
    return out_mi, out_di
